# baseline (speedup 1.0000x reference)
import jax
import jax.numpy as jnp
from jax.experimental import pallas as pl
from jax.experimental.pallas import tpu as pltpu

_TM = 256


def _gate_kernel(x_ref, w_hbm, o_ref, w_buf, w_sem):
    @pl.when(pl.program_id(0) == 0)
    def _load_w():
        copy = pltpu.make_async_copy(w_hbm, w_buf, w_sem)
        copy.start()
        copy.wait()

    dims = (((1,), (1,)), ((), ()))
    o_ref[...] = jax.lax.dot_general(x_ref[...], w_buf[...], dims,
                                     preferred_element_type=jnp.float32)


def kernel(x, W_gate):
    t, d = x.shape
    e = W_gate.shape[0]
    return pl.pallas_call(
        _gate_kernel,
        grid=(t // _TM,),
        in_specs=[
            pl.BlockSpec((_TM, d), lambda i: (i, 0)),
            pl.BlockSpec(memory_space=pl.ANY),
        ],
        out_specs=pl.BlockSpec((_TM, e), lambda i: (i, 0)),
        out_shape=jax.ShapeDtypeStruct((t, e), jnp.float32),
        scratch_shapes=[
            pltpu.VMEM((e, d), jnp.float32),
            pltpu.SemaphoreType.DMA,
        ],
        compiler_params=pltpu.CompilerParams(
            dimension_semantics=(pltpu.ARBITRARY,),
        ),
    )(x, W_gate)


# speedup vs baseline: 1.1706x; 1.1706x over previous
import jax
import jax.numpy as jnp
from jax.experimental import pallas as pl
from jax.experimental.pallas import tpu as pltpu

_TM = 512


def _gate_kernel(x_ref, w_hbm, o_ref, w_buf, w_sem):
    @pl.when(pl.program_id(0) == 0)
    def _load_w():
        copy = pltpu.make_async_copy(w_hbm, w_buf, w_sem)
        copy.start()
        copy.wait()

    dims = (((1,), (0,)), ((), ()))
    o_ref[...] = jax.lax.dot_general(x_ref[...], w_buf[...], dims,
                                     preferred_element_type=jnp.float32)


def kernel(x, W_gate):
    t, d = x.shape
    e = W_gate.shape[0]
    wt = W_gate.T
    return pl.pallas_call(
        _gate_kernel,
        grid=(t // _TM,),
        in_specs=[
            pl.BlockSpec((_TM, d), lambda i: (i, 0)),
            pl.BlockSpec(memory_space=pl.ANY),
        ],
        out_specs=pl.BlockSpec((_TM, e), lambda i: (i, 0)),
        out_shape=jax.ShapeDtypeStruct((t, e), jnp.float32),
        scratch_shapes=[
            pltpu.VMEM((d, e), jnp.float32),
            pltpu.SemaphoreType.DMA,
        ],
        compiler_params=pltpu.CompilerParams(
            dimension_semantics=(pltpu.ARBITRARY,),
        ),
    )(x, wt)


# speedup vs baseline: 1.1934x; 1.0194x over previous
import jax
import jax.numpy as jnp
from jax.experimental import pallas as pl
from jax.experimental.pallas import tpu as pltpu

_TM = 512


def _gate_kernel(x_ref, w_hbm, o_ref, w_buf, w_sem):
    @pl.when(pl.program_id(0) == 0)
    def _load_w():
        copy = pltpu.make_async_copy(w_hbm, w_buf, w_sem)
        copy.start()
        copy.wait()

    dims = (((1,), (1,)), ((), ()))
    o_ref[...] = jax.lax.dot_general(x_ref[...], w_buf[...], dims,
                                     preferred_element_type=jnp.float32)


def kernel(x, W_gate):
    t, d = x.shape
    e = W_gate.shape[0]
    return pl.pallas_call(
        _gate_kernel,
        grid=(t // _TM,),
        in_specs=[
            pl.BlockSpec((_TM, d), lambda i: (i, 0)),
            pl.BlockSpec(memory_space=pl.ANY),
        ],
        out_specs=pl.BlockSpec((_TM, e), lambda i: (i, 0)),
        out_shape=jax.ShapeDtypeStruct((t, e), jnp.float32),
        scratch_shapes=[
            pltpu.VMEM((e, d), jnp.float32),
            pltpu.SemaphoreType.DMA,
        ],
        compiler_params=pltpu.CompilerParams(
            dimension_semantics=(pltpu.ARBITRARY,),
        ),
    )(x, W_gate)
